# Initial kernel scaffold; baseline (speedup 1.0000x reference)
import jax
import jax.numpy as jnp
from jax.experimental import pallas as pl


def kernel(xyz, points, params):
    raise NotImplementedError("write your pallas kernel here")


# speedup vs baseline: 1.0003x; 1.0003x over previous
import jax
import jax.numpy as jnp
import numpy as np
from jax.experimental import pallas as pl
from jax.experimental.pallas import tpu as pltpu

EPS = 1e-5
NPOINT = 512
NSAMPLE = 32


def _index_points(points, idx):
    return jax.vmap(lambda p, i: p[i])(points, idx)


def _fps(xyz, npoint, key):
    B, N, _ = xyz.shape
    farthest = jax.random.randint(key, (B,), 0, N)
    distance = jnp.full((B, N), 1e10, dtype=xyz.dtype)
    centroids = jnp.zeros((B, npoint), dtype=jnp.int32)

    def body(i, st):
        cent, dist_, far = st
        cent = cent.at[:, i].set(far.astype(jnp.int32))
        c = xyz[jnp.arange(B), far, :][:, None, :]
        d = jnp.sum((xyz - c) ** 2, -1)
        dist_ = jnp.minimum(dist_, d)
        far = jnp.argmax(dist_, -1)
        return (cent, dist_, far)

    centroids, _, _ = jax.lax.fori_loop(0, npoint, body, (centroids, distance, farthest))
    return centroids


def _square_distance(src, dst):
    dist = -2.0 * jnp.matmul(src, dst.transpose(0, 2, 1))
    dist = dist + jnp.sum(src ** 2, -1)[:, :, None]
    dist = dist + jnp.sum(dst ** 2, -1)[:, None, :]
    return dist


def _knn(nsample, xyz, new_xyz):
    sqr = _square_distance(new_xyz, xyz)
    _, idx = jax.lax.top_k(-sqr, nsample)
    return idx


def _bn(x, g, b, axes):
    mean = x.mean(axis=axes, keepdims=True)
    var = x.var(axis=axes, keepdims=True)
    shape = [1] * x.ndim
    shape[1] = -1
    return g.reshape(shape) * (x - mean) / jnp.sqrt(var + EPS) + b.reshape(shape)


def _conv1x1(x, W, b):
    return jnp.einsum('bchw,oc->bohw', x, W) + b[None, :, None, None]


def _relu_bn_kernel(x_ref, g_ref, b_ref, m_ref, v_ref, o_ref):
    g = g_ref[...]
    b = b_ref[...]
    m = m_ref[...]
    v = v_ref[...]
    x = x_ref[...]
    o_ref[...] = jnp.maximum(g * (x - m) / jnp.sqrt(v + EPS) + b, 0.0)


def kernel(xyz, points, params):
    B = xyz.shape[0]
    xyz_p = xyz.transpose(0, 2, 1)
    pts_p = points.transpose(0, 2, 1)
    fps_idx = _fps(xyz_p, NPOINT, jax.random.key(42))
    new_xyz = _index_points(xyz_p, fps_idx)
    idx = _knn(NSAMPLE, xyz_p, new_xyz)
    grouped_xyz = _index_points(xyz_p, idx)
    grouped_xyz_norm = grouped_xyz - new_xyz[:, :, None, :]
    grouped_points = _index_points(pts_p, idx)
    new_points = jnp.concatenate([grouped_xyz_norm, grouped_points], axis=-1)
    new_points = new_points.transpose(0, 3, 2, 1)
    for i in range(3):
        new_points = jax.nn.relu(_bn(_conv1x1(new_points, params['conv%d_w' % i], params['conv%d_b' % i]), params['bn%d_g' % i], params['bn%d_b' % i], (0, 2, 3)))
    w = grouped_xyz_norm.transpose(0, 3, 2, 1)
    for i in range(3):
        w = jax.nn.relu(_bn(_conv1x1(w, params['wconv%d_w' % i], params['wconv%d_b' % i]), params['wbn%d_g' % i], params['wbn%d_b' % i], (0, 2, 3)))
    out = jnp.matmul(new_points.transpose(0, 3, 1, 2), w.transpose(0, 3, 2, 1)).reshape(B, NPOINT, -1)
    out = out @ params['lin_w'].T + params['lin_b']
    out = out.transpose(0, 2, 1)
    m = out.mean(axis=(0, 2))
    v = out.var(axis=(0, 2))
    g = params['bnl_g']
    b = params['bnl_b']
    C = out.shape[1]
    out2 = pl.pallas_call(
        _relu_bn_kernel,
        out_shape=jax.ShapeDtypeStruct(out.shape, out.dtype),
        in_specs=[
            pl.BlockSpec(out.shape, lambda: (0, 0, 0)),
            pl.BlockSpec((1, C, 1), lambda: (0, 0, 0)),
            pl.BlockSpec((1, C, 1), lambda: (0, 0, 0)),
            pl.BlockSpec((1, C, 1), lambda: (0, 0, 0)),
            pl.BlockSpec((1, C, 1), lambda: (0, 0, 0)),
        ],
        out_specs=pl.BlockSpec(out.shape, lambda: (0, 0, 0)),
    )(out, g.reshape(1, C, 1), b.reshape(1, C, 1), m.reshape(1, C, 1), v.reshape(1, C, 1))
    return (new_xyz.transpose(0, 2, 1), out2)


# speedup vs baseline: 1.5748x; 1.5743x over previous
import jax
import jax.numpy as jnp
import numpy as np
from jax.experimental import pallas as pl
from jax.experimental.pallas import tpu as pltpu

EPS = 1e-5
NPOINT = 512
NSAMPLE = 32


def _index_points(points, idx):
    return jax.vmap(lambda p, i: p[i])(points, idx)


def _fps_kernel(x_ref, y_ref, z_ref, far0_ref, idx_ref, cx_ref, cy_ref, cz_ref):
    B, N = x_ref.shape
    iota_l = jax.lax.broadcasted_iota(jnp.int32, (B, N), 1)
    iota_p = jax.lax.broadcasted_iota(jnp.int32, (B, NPOINT), 1)
    x = x_ref[...]
    y = y_ref[...]
    z = z_ref[...]

    def body(i, st):
        dist, far, idx_acc, cx_acc, cy_acc, cz_acc = st
        mask = iota_l == far
        cx = jnp.sum(jnp.where(mask, x, 0.0), axis=1, keepdims=True)
        cy = jnp.sum(jnp.where(mask, y, 0.0), axis=1, keepdims=True)
        cz = jnp.sum(jnp.where(mask, z, 0.0), axis=1, keepdims=True)
        sel = iota_p == i
        idx_acc = jnp.where(sel, far, idx_acc)
        cx_acc = jnp.where(sel, cx, cx_acc)
        cy_acc = jnp.where(sel, cy, cy_acc)
        cz_acc = jnp.where(sel, cz, cz_acc)
        dx = x - cx
        dy = y - cy
        dz = z - cz
        d = (dx * dx + dy * dy) + dz * dz
        dist = jnp.minimum(dist, d)
        m = jnp.max(dist, axis=1, keepdims=True)
        far = jnp.min(jnp.where(dist == m, iota_l, N), axis=1, keepdims=True)
        return (dist, far, idx_acc, cx_acc, cy_acc, cz_acc)

    dist0 = jnp.full((B, N), 1e10, dtype=jnp.float32)
    zp = jnp.zeros((B, NPOINT), dtype=jnp.float32)
    zi = jnp.zeros((B, NPOINT), dtype=jnp.int32)
    _, _, idx_acc, cx_acc, cy_acc, cz_acc = jax.lax.fori_loop(
        0, NPOINT, body, (dist0, far0_ref[...], zi, zp, zp, zp))
    idx_ref[...] = idx_acc
    cx_ref[...] = cx_acc
    cy_ref[...] = cy_acc
    cz_ref[...] = cz_acc


def _fps_pallas(xyz_p):
    B, N, _ = xyz_p.shape
    far0 = jax.random.randint(jax.random.key(42), (B,), 0, N).astype(jnp.int32)[:, None]
    x = xyz_p[:, :, 0]
    y = xyz_p[:, :, 1]
    z = xyz_p[:, :, 2]
    idx, cx, cy, cz = pl.pallas_call(
        _fps_kernel,
        out_shape=(
            jax.ShapeDtypeStruct((B, NPOINT), jnp.int32),
            jax.ShapeDtypeStruct((B, NPOINT), jnp.float32),
            jax.ShapeDtypeStruct((B, NPOINT), jnp.float32),
            jax.ShapeDtypeStruct((B, NPOINT), jnp.float32),
        ),
    )(x, y, z, far0)
    new_xyz = jnp.stack([cx, cy, cz], axis=2)
    return idx, new_xyz


def _square_distance(src, dst):
    dist = -2.0 * jnp.matmul(src, dst.transpose(0, 2, 1))
    dist = dist + jnp.sum(src ** 2, -1)[:, :, None]
    dist = dist + jnp.sum(dst ** 2, -1)[:, None, :]
    return dist


def _knn(nsample, xyz, new_xyz):
    sqr = _square_distance(new_xyz, xyz)
    _, idx = jax.lax.top_k(-sqr, nsample)
    return idx


def _bn(x, g, b, axes):
    mean = x.mean(axis=axes, keepdims=True)
    var = x.var(axis=axes, keepdims=True)
    shape = [1] * x.ndim
    shape[1] = -1
    return g.reshape(shape) * (x - mean) / jnp.sqrt(var + EPS) + b.reshape(shape)


def _conv1x1(x, W, b):
    return jnp.einsum('bchw,oc->bohw', x, W) + b[None, :, None, None]


def _relu_bn_kernel(x_ref, g_ref, b_ref, m_ref, v_ref, o_ref):
    g = g_ref[...]
    b = b_ref[...]
    m = m_ref[...]
    v = v_ref[...]
    x = x_ref[...]
    o_ref[...] = jnp.maximum(g * (x - m) / jnp.sqrt(v + EPS) + b, 0.0)


def kernel(xyz, points, params):
    B = xyz.shape[0]
    xyz_p = xyz.transpose(0, 2, 1)
    pts_p = points.transpose(0, 2, 1)
    fps_idx, new_xyz = _fps_pallas(xyz_p)
    idx = _knn(NSAMPLE, xyz_p, new_xyz)
    grouped_xyz = _index_points(xyz_p, idx)
    grouped_xyz_norm = grouped_xyz - new_xyz[:, :, None, :]
    grouped_points = _index_points(pts_p, idx)
    new_points = jnp.concatenate([grouped_xyz_norm, grouped_points], axis=-1)
    new_points = new_points.transpose(0, 3, 2, 1)
    for i in range(3):
        new_points = jax.nn.relu(_bn(_conv1x1(new_points, params['conv%d_w' % i], params['conv%d_b' % i]), params['bn%d_g' % i], params['bn%d_b' % i], (0, 2, 3)))
    w = grouped_xyz_norm.transpose(0, 3, 2, 1)
    for i in range(3):
        w = jax.nn.relu(_bn(_conv1x1(w, params['wconv%d_w' % i], params['wconv%d_b' % i]), params['wbn%d_g' % i], params['wbn%d_b' % i], (0, 2, 3)))
    out = jnp.matmul(new_points.transpose(0, 3, 1, 2), w.transpose(0, 3, 2, 1)).reshape(B, NPOINT, -1)
    out = out @ params['lin_w'].T + params['lin_b']
    out = out.transpose(0, 2, 1)
    m = out.mean(axis=(0, 2))
    v = out.var(axis=(0, 2))
    g = params['bnl_g']
    b = params['bnl_b']
    C = out.shape[1]
    out2 = pl.pallas_call(
        _relu_bn_kernel,
        out_shape=jax.ShapeDtypeStruct(out.shape, out.dtype),
        in_specs=[
            pl.BlockSpec(out.shape, lambda: (0, 0, 0)),
            pl.BlockSpec((1, C, 1), lambda: (0, 0, 0)),
            pl.BlockSpec((1, C, 1), lambda: (0, 0, 0)),
            pl.BlockSpec((1, C, 1), lambda: (0, 0, 0)),
            pl.BlockSpec((1, C, 1), lambda: (0, 0, 0)),
        ],
        out_specs=pl.BlockSpec(out.shape, lambda: (0, 0, 0)),
    )(out, g.reshape(1, C, 1), b.reshape(1, C, 1), m.reshape(1, C, 1), v.reshape(1, C, 1))
    return (new_xyz.transpose(0, 2, 1), out2)


# speedup vs baseline: 2.3560x; 1.4960x over previous
import jax
import jax.numpy as jnp
import numpy as np
from jax.experimental import pallas as pl
from jax.experimental.pallas import tpu as pltpu

EPS = 1e-5
NPOINT = 512
NSAMPLE = 32


def _index_points(points, idx):
    return jax.vmap(lambda p, i: p[i])(points, idx)


def _fps_kernel(x_ref, y_ref, z_ref, far0_ref, idx_ref, cx_ref, cy_ref, cz_ref):
    B, N = x_ref.shape
    iota_l = jax.lax.broadcasted_iota(jnp.int32, (B, N), 1)
    iota_p = jax.lax.broadcasted_iota(jnp.int32, (B, NPOINT), 1)
    x = x_ref[...]
    y = y_ref[...]
    z = z_ref[...]

    def body(i, st):
        dist, far, idx_acc, cx_acc, cy_acc, cz_acc = st
        mask = iota_l == far
        cx = jnp.sum(jnp.where(mask, x, 0.0), axis=1, keepdims=True)
        cy = jnp.sum(jnp.where(mask, y, 0.0), axis=1, keepdims=True)
        cz = jnp.sum(jnp.where(mask, z, 0.0), axis=1, keepdims=True)
        sel = iota_p == i
        idx_acc = jnp.where(sel, far, idx_acc)
        cx_acc = jnp.where(sel, cx, cx_acc)
        cy_acc = jnp.where(sel, cy, cy_acc)
        cz_acc = jnp.where(sel, cz, cz_acc)
        dx = x - cx
        dy = y - cy
        dz = z - cz
        d = (dx * dx + dy * dy) + dz * dz
        dist = jnp.minimum(dist, d)
        m = jnp.max(dist, axis=1, keepdims=True)
        far = jnp.min(jnp.where(dist == m, iota_l, N), axis=1, keepdims=True)
        return (dist, far, idx_acc, cx_acc, cy_acc, cz_acc)

    dist0 = jnp.full((B, N), 1e10, dtype=jnp.float32)
    zp = jnp.zeros((B, NPOINT), dtype=jnp.float32)
    zi = jnp.zeros((B, NPOINT), dtype=jnp.int32)
    _, _, idx_acc, cx_acc, cy_acc, cz_acc = jax.lax.fori_loop(
        0, NPOINT, body, (dist0, far0_ref[...], zi, zp, zp, zp))
    idx_ref[...] = idx_acc
    cx_ref[...] = cx_acc
    cy_ref[...] = cy_acc
    cz_ref[...] = cz_acc


def _fps_pallas(xyz_p):
    B, N, _ = xyz_p.shape
    far0 = jax.random.randint(jax.random.key(42), (B,), 0, N).astype(jnp.int32)[:, None]
    x = xyz_p[:, :, 0]
    y = xyz_p[:, :, 1]
    z = xyz_p[:, :, 2]
    idx, cx, cy, cz = pl.pallas_call(
        _fps_kernel,
        out_shape=(
            jax.ShapeDtypeStruct((B, NPOINT), jnp.int32),
            jax.ShapeDtypeStruct((B, NPOINT), jnp.float32),
            jax.ShapeDtypeStruct((B, NPOINT), jnp.float32),
            jax.ShapeDtypeStruct((B, NPOINT), jnp.float32),
        ),
    )(x, y, z, far0)
    new_xyz = jnp.stack([cx, cy, cz], axis=2)
    return idx, new_xyz


def _knn_kernel(nx8_ref, xyzT8_ref, idx_ref):
    M, N = 512, 4096
    nx8 = nx8_ref[0]
    xyzT8 = xyzT8_ref[0]
    mm = jax.lax.dot_general(nx8, xyzT8, (((1,), (0,)), ((), ())),
                             preferred_element_type=jnp.float32)
    sqr = -2.0 * mm
    sqr = sqr + jnp.sum(nx8 * nx8, axis=1, keepdims=True)
    sqr = sqr + jnp.sum(xyzT8 * xyzT8, axis=0, keepdims=True)
    iota_l = jax.lax.broadcasted_iota(jnp.int32, (M, N), 1)
    cols = []
    for _ in range(NSAMPLE):
        m = jnp.min(sqr, axis=1, keepdims=True)
        sel = jnp.min(jnp.where(sqr == m, iota_l, N), axis=1, keepdims=True)
        cols.append(sel)
        sqr = jnp.where(iota_l == sel, jnp.inf, sqr)
    idx_ref[0] = jnp.concatenate(cols, axis=1)


def _knn_pallas(xyz_p, new_xyz):
    B, N, _ = xyz_p.shape
    nx8 = jnp.concatenate([new_xyz, jnp.zeros((B, NPOINT, 5), jnp.float32)], axis=2)
    xyzT8 = jnp.concatenate([xyz_p.transpose(0, 2, 1), jnp.zeros((B, 5, N), jnp.float32)], axis=1)
    idx = pl.pallas_call(
        _knn_kernel,
        grid=(B,),
        in_specs=[
            pl.BlockSpec((1, NPOINT, 8), lambda b: (b, 0, 0)),
            pl.BlockSpec((1, 8, N), lambda b: (b, 0, 0)),
        ],
        out_specs=pl.BlockSpec((1, NPOINT, NSAMPLE), lambda b: (b, 0, 0)),
        out_shape=jax.ShapeDtypeStruct((B, NPOINT, NSAMPLE), jnp.int32),
    )(nx8, xyzT8)
    return idx


def _bn(x, g, b, axes):
    mean = x.mean(axis=axes, keepdims=True)
    var = x.var(axis=axes, keepdims=True)
    shape = [1] * x.ndim
    shape[1] = -1
    return g.reshape(shape) * (x - mean) / jnp.sqrt(var + EPS) + b.reshape(shape)


def _conv1x1(x, W, b):
    return jnp.einsum('bchw,oc->bohw', x, W) + b[None, :, None, None]


def _relu_bn_kernel(x_ref, g_ref, b_ref, m_ref, v_ref, o_ref):
    g = g_ref[...]
    b = b_ref[...]
    m = m_ref[...]
    v = v_ref[...]
    x = x_ref[...]
    o_ref[...] = jnp.maximum(g * (x - m) / jnp.sqrt(v + EPS) + b, 0.0)


def kernel(xyz, points, params):
    B = xyz.shape[0]
    xyz_p = xyz.transpose(0, 2, 1)
    pts_p = points.transpose(0, 2, 1)
    fps_idx, new_xyz = _fps_pallas(xyz_p)
    idx = _knn_pallas(xyz_p, new_xyz)
    grouped_xyz = _index_points(xyz_p, idx)
    grouped_xyz_norm = grouped_xyz - new_xyz[:, :, None, :]
    grouped_points = _index_points(pts_p, idx)
    new_points = jnp.concatenate([grouped_xyz_norm, grouped_points], axis=-1)
    new_points = new_points.transpose(0, 3, 2, 1)
    for i in range(3):
        new_points = jax.nn.relu(_bn(_conv1x1(new_points, params['conv%d_w' % i], params['conv%d_b' % i]), params['bn%d_g' % i], params['bn%d_b' % i], (0, 2, 3)))
    w = grouped_xyz_norm.transpose(0, 3, 2, 1)
    for i in range(3):
        w = jax.nn.relu(_bn(_conv1x1(w, params['wconv%d_w' % i], params['wconv%d_b' % i]), params['wbn%d_g' % i], params['wbn%d_b' % i], (0, 2, 3)))
    out = jnp.matmul(new_points.transpose(0, 3, 1, 2), w.transpose(0, 3, 2, 1)).reshape(B, NPOINT, -1)
    out = out @ params['lin_w'].T + params['lin_b']
    out = out.transpose(0, 2, 1)
    m = out.mean(axis=(0, 2))
    v = out.var(axis=(0, 2))
    g = params['bnl_g']
    b = params['bnl_b']
    C = out.shape[1]
    out2 = pl.pallas_call(
        _relu_bn_kernel,
        out_shape=jax.ShapeDtypeStruct(out.shape, out.dtype),
        in_specs=[
            pl.BlockSpec(out.shape, lambda: (0, 0, 0)),
            pl.BlockSpec((1, C, 1), lambda: (0, 0, 0)),
            pl.BlockSpec((1, C, 1), lambda: (0, 0, 0)),
            pl.BlockSpec((1, C, 1), lambda: (0, 0, 0)),
            pl.BlockSpec((1, C, 1), lambda: (0, 0, 0)),
        ],
        out_specs=pl.BlockSpec(out.shape, lambda: (0, 0, 0)),
    )(out, g.reshape(1, C, 1), b.reshape(1, C, 1), m.reshape(1, C, 1), v.reshape(1, C, 1))
    return (new_xyz.transpose(0, 2, 1), out2)


# speedup vs baseline: 8.0212x; 3.4045x over previous
import functools

import jax
import jax.numpy as jnp
import numpy as np
from jax import lax
from jax.experimental import pallas as pl
from jax.experimental.pallas import tpu as pltpu
from jax.experimental.pallas import tpu_sc as plsc

EPS = 1e-5
NPOINT = 512
NSAMPLE = 32


def _index_points(points, idx):
    return jax.vmap(lambda p, i: p[i])(points, idx)


def _fps_kernel(x_ref, y_ref, z_ref, far0_ref, idx_ref, cx_ref, cy_ref, cz_ref):
    B, N = x_ref.shape
    iota_l = jax.lax.broadcasted_iota(jnp.int32, (B, N), 1)
    iota_p = jax.lax.broadcasted_iota(jnp.int32, (B, NPOINT), 1)
    x = x_ref[...]
    y = y_ref[...]
    z = z_ref[...]

    def body(i, st):
        dist, far, idx_acc, cx_acc, cy_acc, cz_acc = st
        mask = iota_l == far
        cx = jnp.sum(jnp.where(mask, x, 0.0), axis=1, keepdims=True)
        cy = jnp.sum(jnp.where(mask, y, 0.0), axis=1, keepdims=True)
        cz = jnp.sum(jnp.where(mask, z, 0.0), axis=1, keepdims=True)
        sel = iota_p == i
        idx_acc = jnp.where(sel, far, idx_acc)
        cx_acc = jnp.where(sel, cx, cx_acc)
        cy_acc = jnp.where(sel, cy, cy_acc)
        cz_acc = jnp.where(sel, cz, cz_acc)
        dx = x - cx
        dy = y - cy
        dz = z - cz
        d = (dx * dx + dy * dy) + dz * dz
        dist = jnp.minimum(dist, d)
        m = jnp.max(dist, axis=1, keepdims=True)
        far = jnp.min(jnp.where(dist == m, iota_l, N), axis=1, keepdims=True)
        return (dist, far, idx_acc, cx_acc, cy_acc, cz_acc)

    dist0 = jnp.full((B, N), 1e10, dtype=jnp.float32)
    zp = jnp.zeros((B, NPOINT), dtype=jnp.float32)
    zi = jnp.zeros((B, NPOINT), dtype=jnp.int32)
    _, _, idx_acc, cx_acc, cy_acc, cz_acc = jax.lax.fori_loop(
        0, NPOINT, body, (dist0, far0_ref[...], zi, zp, zp, zp))
    idx_ref[...] = idx_acc
    cx_ref[...] = cx_acc
    cy_ref[...] = cy_acc
    cz_ref[...] = cz_acc


def _fps_pallas(xyz_p):
    B, N, _ = xyz_p.shape
    far0 = jax.random.randint(jax.random.key(42), (B,), 0, N).astype(jnp.int32)[:, None]
    x = xyz_p[:, :, 0]
    y = xyz_p[:, :, 1]
    z = xyz_p[:, :, 2]
    idx, cx, cy, cz = pl.pallas_call(
        _fps_kernel,
        out_shape=(
            jax.ShapeDtypeStruct((B, NPOINT), jnp.int32),
            jax.ShapeDtypeStruct((B, NPOINT), jnp.float32),
            jax.ShapeDtypeStruct((B, NPOINT), jnp.float32),
            jax.ShapeDtypeStruct((B, NPOINT), jnp.float32),
        ),
    )(x, y, z, far0)
    new_xyz = jnp.stack([cx, cy, cz], axis=2)
    return idx, new_xyz


def _knn_kernel(nx8_ref, xyzT8_ref, idx_ref):
    M, N = 512, 4096
    nx8 = nx8_ref[0]
    xyzT8 = xyzT8_ref[0]
    mm = jax.lax.dot_general(nx8, xyzT8, (((1,), (0,)), ((), ())),
                             preferred_element_type=jnp.float32)
    sqr = -2.0 * mm
    sqr = sqr + jnp.sum(nx8 * nx8, axis=1, keepdims=True)
    sqr = sqr + jnp.sum(xyzT8 * xyzT8, axis=0, keepdims=True)
    iota_l = jax.lax.broadcasted_iota(jnp.int32, (M, N), 1)
    cols = []
    for _ in range(NSAMPLE):
        m = jnp.min(sqr, axis=1, keepdims=True)
        sel = jnp.min(jnp.where(sqr == m, iota_l, N), axis=1, keepdims=True)
        cols.append(sel)
        sqr = jnp.where(iota_l == sel, jnp.inf, sqr)
    idx_ref[0] = jnp.concatenate(cols, axis=1)


def _knn_pallas(xyz_p, new_xyz):
    B, N, _ = xyz_p.shape
    nx8 = jnp.concatenate([new_xyz, jnp.zeros((B, NPOINT, 5), jnp.float32)], axis=2)
    xyzT8 = jnp.concatenate([xyz_p.transpose(0, 2, 1), jnp.zeros((B, 5, N), jnp.float32)], axis=1)
    idx = pl.pallas_call(
        _knn_kernel,
        grid=(B,),
        in_specs=[
            pl.BlockSpec((1, NPOINT, 8), lambda b: (b, 0, 0)),
            pl.BlockSpec((1, 8, N), lambda b: (b, 0, 0)),
        ],
        out_specs=pl.BlockSpec((1, NPOINT, NSAMPLE), lambda b: (b, 0, 0)),
        out_shape=jax.ShapeDtypeStruct((B, NPOINT, NSAMPLE), jnp.int32),
    )(nx8, xyzT8)
    return idx


def _sc_gather_rows(table, gidx, ncols):
    NR = gidx.shape[0]
    NW = 32
    rows_per_w = NR // NW
    CHUNK = 128
    n_chunks = rows_per_w // CHUNK
    mesh = plsc.VectorSubcoreMesh(core_axis_name="c", subcore_axis_name="s")

    @functools.partial(
        pl.kernel,
        mesh=mesh,
        out_type=jax.ShapeDtypeStruct((NR, ncols), jnp.float32),
        scratch_types=[
            pltpu.VMEM((CHUNK,), jnp.int32),
            pltpu.VMEM((CHUNK, ncols), jnp.float32),
            pltpu.SemaphoreType.DMA,
        ],
        compiler_params=pltpu.CompilerParams(use_tc_tiling_on_sc=False),
    )
    def k(table_hbm, gidx_hbm, out_hbm, idx_v, rows_v, sem):
        wid = lax.axis_index("s") * 2 + lax.axis_index("c")
        base = wid * rows_per_w

        def chunk_body(ci, _):
            cb = base + ci * CHUNK
            pltpu.sync_copy(gidx_hbm.at[pl.ds(cb, CHUNK)], idx_v)
            pltpu.async_copy(table_hbm.at[idx_v], rows_v, sem).wait()
            pltpu.sync_copy(rows_v, out_hbm.at[pl.ds(cb, CHUNK)])
            return 0

        lax.fori_loop(0, n_chunks, chunk_body, 0)

    return k(table, gidx)


def _bn(x, g, b, axes):
    mean = x.mean(axis=axes, keepdims=True)
    var = x.var(axis=axes, keepdims=True)
    shape = [1] * x.ndim
    shape[1] = -1
    return g.reshape(shape) * (x - mean) / jnp.sqrt(var + EPS) + b.reshape(shape)


def _conv1x1(x, W, b):
    return jnp.einsum('bchw,oc->bohw', x, W) + b[None, :, None, None]


def _relu_bn_kernel(x_ref, g_ref, b_ref, m_ref, v_ref, o_ref):
    g = g_ref[...]
    b = b_ref[...]
    m = m_ref[...]
    v = v_ref[...]
    x = x_ref[...]
    o_ref[...] = jnp.maximum(g * (x - m) / jnp.sqrt(v + EPS) + b, 0.0)


def kernel(xyz, points, params):
    B = xyz.shape[0]
    xyz_p = xyz.transpose(0, 2, 1)
    pts_p = points.transpose(0, 2, 1)
    fps_idx, new_xyz = _fps_pallas(xyz_p)
    idx = _knn_pallas(xyz_p, new_xyz)
    N = xyz_p.shape[1]
    table = jnp.concatenate(
        [xyz_p, pts_p, jnp.zeros((B, N, 13), jnp.float32)], axis=2).reshape(B * N, 144)
    gidx = (idx + (jnp.arange(B, dtype=jnp.int32) * N)[:, None, None]).reshape(-1)
    G = _sc_gather_rows(table, gidx, 144)
    grouped_xyz = G[:, 0:3].reshape(B, NPOINT, NSAMPLE, 3)
    grouped_points = G[:, 3:131].reshape(B, NPOINT, NSAMPLE, 128)
    grouped_xyz_norm = grouped_xyz - new_xyz[:, :, None, :]
    new_points = jnp.concatenate([grouped_xyz_norm, grouped_points], axis=-1)
    new_points = new_points.transpose(0, 3, 2, 1)
    for i in range(3):
        new_points = jax.nn.relu(_bn(_conv1x1(new_points, params['conv%d_w' % i], params['conv%d_b' % i]), params['bn%d_g' % i], params['bn%d_b' % i], (0, 2, 3)))
    w = grouped_xyz_norm.transpose(0, 3, 2, 1)
    for i in range(3):
        w = jax.nn.relu(_bn(_conv1x1(w, params['wconv%d_w' % i], params['wconv%d_b' % i]), params['wbn%d_g' % i], params['wbn%d_b' % i], (0, 2, 3)))
    out = jnp.matmul(new_points.transpose(0, 3, 1, 2), w.transpose(0, 3, 2, 1)).reshape(B, NPOINT, -1)
    out = out @ params['lin_w'].T + params['lin_b']
    out = out.transpose(0, 2, 1)
    m = out.mean(axis=(0, 2))
    v = out.var(axis=(0, 2))
    g = params['bnl_g']
    b = params['bnl_b']
    C = out.shape[1]
    out2 = pl.pallas_call(
        _relu_bn_kernel,
        out_shape=jax.ShapeDtypeStruct(out.shape, out.dtype),
        in_specs=[
            pl.BlockSpec(out.shape, lambda: (0, 0, 0)),
            pl.BlockSpec((1, C, 1), lambda: (0, 0, 0)),
            pl.BlockSpec((1, C, 1), lambda: (0, 0, 0)),
            pl.BlockSpec((1, C, 1), lambda: (0, 0, 0)),
            pl.BlockSpec((1, C, 1), lambda: (0, 0, 0)),
        ],
        out_specs=pl.BlockSpec(out.shape, lambda: (0, 0, 0)),
    )(out, g.reshape(1, C, 1), b.reshape(1, C, 1), m.reshape(1, C, 1), v.reshape(1, C, 1))
    return (new_xyz.transpose(0, 2, 1), out2)
